# baseline (speedup 1.0000x reference)
import functools

import jax
import jax.numpy as jnp
from jax import lax
from jax.experimental import pallas as pl
from jax.experimental.pallas import tpu as pltpu
from jax.experimental.pallas import tpu_sc as plsc

_N = 10000
_R = 16
_D = 128
_E = 320000
_NR = _N * _R
_NC = 2
_NS = 16
_CHUNK = 2000
_B = 80
_ROWS = _CHUNK // _B
_P1_EDGES = _E // _NS
_P3_EDGES = _E // (_NC * _NS)
_NODES_T = 632
_LAST_T = _N - (_NS - 1) * _NODES_T
_DEG_T = _NR // _NS


def _transform_body(h_ref, w_ref, o_ref):
    o_ref[0] = jnp.dot(h_ref[...], w_ref[0], preferred_element_type=jnp.float32)


def _final_body(p_ref, h_ref, ws_ref, o_ref):
    o_ref[...] = jax.nn.relu(
        p_ref[0] + p_ref[1]
        + jnp.dot(h_ref[...], ws_ref[...], preferred_element_type=jnp.float32)
    )


_BN = 1000


def _transform(h, W_rel):
    return pl.pallas_call(
        _transform_body,
        grid=(_R, _N // _BN),
        in_specs=[
            pl.BlockSpec((_BN, _D), lambda r, nb: (nb, 0)),
            pl.BlockSpec((1, _D, _D), lambda r, nb: (r, 0, 0)),
        ],
        out_specs=pl.BlockSpec((1, _BN, _D), lambda r, nb: (r, nb, 0)),
        out_shape=jax.ShapeDtypeStruct((_R, _N, _D), jnp.float32),
    )(h, W_rel)


def _final(partials, h, W_self):
    return pl.pallas_call(
        _final_body,
        grid=(_N // _BN,),
        in_specs=[
            pl.BlockSpec((_NC, _BN, _D), lambda nb: (0, nb, 0)),
            pl.BlockSpec((_BN, _D), lambda nb: (nb, 0)),
            pl.BlockSpec((_D, _D), lambda nb: (0, 0)),
        ],
        out_specs=pl.BlockSpec((_BN, _D), lambda nb: (nb, 0)),
        out_shape=jax.ShapeDtypeStruct((_N, _D), jnp.float32),
    )(partials, h, W_self)


_sc_mesh = plsc.VectorSubcoreMesh(core_axis_name="c", subcore_axis_name="s")


@functools.partial(
    pl.kernel,
    mesh=_sc_mesh,
    out_type=jax.ShapeDtypeStruct((_NC, _N, _D), jnp.float32),
    scratch_types=[
        pltpu.VMEM_SHARED((_N, _D), jnp.float32),
        pltpu.VMEM_SHARED((_NR,), jnp.float32),
        pltpu.VMEM((_CHUNK,), jnp.int32),
        pltpu.VMEM((_CHUNK,), jnp.int32),
        pltpu.VMEM((_CHUNK,), jnp.int32),
        pltpu.VMEM((_ROWS, _B), jnp.int32),
        pltpu.VMEM((2, _B), jnp.int32),
        pltpu.VMEM((2, _B), jnp.int32),
        pltpu.VMEM((2, _B), jnp.int32),
        pltpu.VMEM((2, _B), jnp.float32),
        pltpu.VMEM((2, _B, _D), jnp.float32),
        pltpu.VMEM((_CHUNK,), jnp.float32),
        pltpu.VMEM((_B,), jnp.float32),
        pltpu.SemaphoreType.DMA,
        pltpu.SemaphoreType.DMA,
        pltpu.SemaphoreType.DMA,
    ],
)
def _sc_kernel(src_hbm, rel_hbm, dst_hbm, table_hbm, zagg_hbm,
               out_hbm,
               agg_sh, deg_sh, relv, dstv, srcv, p1pair, pair2, didx2, gidx2,
               norm2, rows2, degv, ones, sem_g, sem_n, sem_s):
    c = lax.axis_index("c")
    s = lax.axis_index("s")

    @pl.when(s < _NS - 1)
    def _():
        pltpu.sync_copy(zagg_hbm.at[pl.ds(s * _NODES_T, _NODES_T)],
                        agg_sh.at[pl.ds(s * _NODES_T, _NODES_T)])

    @pl.when(s == _NS - 1)
    def _():
        pltpu.sync_copy(zagg_hbm.at[pl.ds((_NS - 1) * _NODES_T, _LAST_T)],
                        agg_sh.at[pl.ds((_NS - 1) * _NODES_T, _LAST_T)])

    def zero_vec(j, carry):
        degv[pl.ds(j * 16, 16)] = jnp.zeros((16,), jnp.float32)
        return carry

    lax.fori_loop(0, _CHUNK // 16, zero_vec, 0)

    def zero_deg(k, carry):
        pltpu.sync_copy(degv, deg_sh.at[pl.ds(s * _DEG_T + k * _CHUNK, _CHUNK)])
        return carry

    lax.fori_loop(0, _DEG_T // _CHUNK, zero_deg, 0)
    for q in range(_B // 16):
        ones[pl.ds(q * 16, 16)] = jnp.full((16,), 1.0, jnp.float32)
    plsc.subcore_barrier()

    def p1_chunk(k, carry):
        e0 = s * _P1_EDGES + k * _CHUNK
        pltpu.sync_copy(rel_hbm.at[pl.ds(e0, _CHUNK)], relv)
        pltpu.sync_copy(dst_hbm.at[pl.ds(e0, _CHUNK)], dstv)

        def p1_row(r, carry2):
            for q in range(_B // 16):
                o = r * _B + q * 16
                pr = dstv[pl.ds(o, 16)] * _R + relv[pl.ds(o, 16)]
                p1pair[r, pl.ds(q * 16, 16)] = pr
            return carry2

        lax.fori_loop(0, _ROWS, p1_row, 0)

        def p1_scat(r, carry2):
            pltpu.async_copy(ones, deg_sh.at[p1pair.at[r]], sem_s, add=True)
            return carry2

        lax.fori_loop(0, _ROWS, p1_scat, 0)

        def p1_drain(r, carry2):
            pltpu.make_async_copy(ones, deg_sh.at[p1pair.at[r]], sem_s).wait()
            return carry2

        lax.fori_loop(0, _ROWS, p1_drain, 0)
        return carry

    lax.fori_loop(0, _P1_EDGES // _CHUNK, p1_chunk, 0)
    plsc.subcore_barrier()

    base = s * _DEG_T

    def p2_chunk(k, carry):
        pltpu.sync_copy(deg_sh.at[pl.ds(base + k * _CHUNK, _CHUNK)], degv)

        def p2_vec(j, carry2):
            v = degv[pl.ds(j * 16, 16)]
            degv[pl.ds(j * 16, 16)] = 1.0 / jnp.maximum(v, 1.0)
            return carry2

        lax.fori_loop(0, _CHUNK // 16, p2_vec, 0)
        pltpu.sync_copy(degv, deg_sh.at[pl.ds(base + k * _CHUNK, _CHUNK)])
        return carry

    lax.fori_loop(0, _DEG_T // _CHUNK, p2_chunk, 0)
    plsc.subcore_barrier()

    def p3_idx(r, b):
        for q in range(_B // 16):
            o = r * _B + q * 16
            rl = relv[pl.ds(o, 16)]
            sr = srcv[pl.ds(o, 16)]
            dd = dstv[pl.ds(o, 16)]
            gidx2[b, pl.ds(q * 16, 16)] = rl * _N + sr
            pair2[b, pl.ds(q * 16, 16)] = dd * _R + rl
            didx2[b, pl.ds(q * 16, 16)] = dd

    def p3_chunk(k, carry):
        e0 = c * (_E // _NC) + s * _P3_EDGES + k * _CHUNK
        pltpu.sync_copy(src_hbm.at[pl.ds(e0, _CHUNK)], srcv)
        pltpu.sync_copy(rel_hbm.at[pl.ds(e0, _CHUNK)], relv)
        pltpu.sync_copy(dst_hbm.at[pl.ds(e0, _CHUNK)], dstv)

        p3_idx(0, 0)
        pltpu.async_copy(table_hbm.at[gidx2.at[0]], rows2.at[0], sem_g)
        pltpu.async_copy(deg_sh.at[pair2.at[0]], norm2.at[0], sem_n)

        def p3_row(r, carry2):
            b = lax.rem(r, 2)

            @pl.when(r >= 1)
            def _():
                pltpu.make_async_copy(rows2.at[1 - b], agg_sh.at[didx2.at[0]],
                                      sem_s).wait()

            @pl.when(r + 1 < _ROWS)
            def _():
                p3_idx(r + 1, 1 - b)
                pltpu.async_copy(table_hbm.at[gidx2.at[1 - b]],
                                 rows2.at[1 - b], sem_g)
                pltpu.async_copy(deg_sh.at[pair2.at[1 - b]],
                                 norm2.at[1 - b], sem_n)

            pltpu.make_async_copy(table_hbm.at[gidx2.at[b]], rows2.at[b],
                                  sem_g).wait()
            pltpu.make_async_copy(deg_sh.at[pair2.at[b]], norm2.at[b],
                                  sem_n).wait()

            def scale(g, carry3):
                n16 = norm2[b, pl.ds(g * 16, 16)]
                for l in range(16):
                    nb = jnp.full((16,), n16[l], jnp.float32)
                    j = g * 16 + l
                    for q in range(_D // 16):
                        v = rows2[b, j, pl.ds(q * 16, 16)]
                        rows2[b, j, pl.ds(q * 16, 16)] = v * nb
                return carry3

            lax.fori_loop(0, _B // 16, scale, 0)
            pltpu.async_copy(rows2.at[b], agg_sh.at[didx2.at[b]], sem_s,
                             add=True)
            return carry2

        lax.fori_loop(0, _ROWS, p3_row, 0)
        pltpu.make_async_copy(rows2.at[0], agg_sh.at[didx2.at[0]], sem_s).wait()
        return carry

    lax.fori_loop(0, _P3_EDGES // _CHUNK, p3_chunk, 0)
    plsc.subcore_barrier()

    @pl.when(s < _NS - 1)
    def _():
        pltpu.sync_copy(agg_sh.at[pl.ds(s * _NODES_T, _NODES_T)],
                        out_hbm.at[c, pl.ds(s * _NODES_T, _NODES_T)])

    @pl.when(s == _NS - 1)
    def _():
        pltpu.sync_copy(agg_sh.at[pl.ds((_NS - 1) * _NODES_T, _LAST_T)],
                        out_hbm.at[c, pl.ds((_NS - 1) * _NODES_T, _LAST_T)])


def kernel(edges, h_input, W_rel, W_self):
    src = edges[:, 0].astype(jnp.int32)
    rel = (edges[:, 1] % _R).astype(jnp.int32)
    dst = edges[:, 2].astype(jnp.int32)
    transformed = _transform(h_input, W_rel).reshape(_R * _N, _D)
    zagg = jnp.zeros((_N, _D), jnp.float32)
    partials = _sc_kernel(src, rel, dst, transformed, zagg)
    return _final(partials, h_input, W_self)


# speedup vs baseline: 1.0235x; 1.0235x over previous
import functools

import jax
import jax.numpy as jnp
from jax import lax
from jax.experimental import pallas as pl
from jax.experimental.pallas import tpu as pltpu
from jax.experimental.pallas import tpu_sc as plsc

_N = 10000
_R = 16
_D = 128
_E = 320000
_NR = _N * _R
_NC = 2
_NS = 16
_CHUNK = 2000
_B = 80
_ROWS = _CHUNK // _B
_P1_EDGES = _E // _NS
_P3_EDGES = _E // (_NC * _NS)
_NODES_T = 632
_LAST_T = _N - (_NS - 1) * _NODES_T
_DEG_T = _NR // _NS


def _transform_body(h_ref, w_ref, o_ref):
    o_ref[0] = jnp.dot(h_ref[...], w_ref[0], preferred_element_type=jnp.float32)


def _final_body(p_ref, h_ref, ws_ref, o_ref):
    o_ref[...] = jax.nn.relu(
        p_ref[0] + p_ref[1]
        + jnp.dot(h_ref[...], ws_ref[...], preferred_element_type=jnp.float32)
    )


_BN = 1000


def _transform(h, W_rel):
    return pl.pallas_call(
        _transform_body,
        grid=(_R, _N // _BN),
        in_specs=[
            pl.BlockSpec((_BN, _D), lambda r, nb: (nb, 0)),
            pl.BlockSpec((1, _D, _D), lambda r, nb: (r, 0, 0)),
        ],
        out_specs=pl.BlockSpec((1, _BN, _D), lambda r, nb: (r, nb, 0)),
        out_shape=jax.ShapeDtypeStruct((_R, _N, _D), jnp.float32),
    )(h, W_rel)


def _final(partials, h, W_self):
    return pl.pallas_call(
        _final_body,
        grid=(_N // _BN,),
        in_specs=[
            pl.BlockSpec((_NC, _BN, _D), lambda nb: (0, nb, 0)),
            pl.BlockSpec((_BN, _D), lambda nb: (nb, 0)),
            pl.BlockSpec((_D, _D), lambda nb: (0, 0)),
        ],
        out_specs=pl.BlockSpec((_BN, _D), lambda nb: (nb, 0)),
        out_shape=jax.ShapeDtypeStruct((_N, _D), jnp.float32),
    )(partials, h, W_self)


def _norm_body(d_ref, o_ref):
    o_ref[...] = 1.0 / jnp.maximum(d_ref[0] + d_ref[1], 1.0)


def _norm(degp):
    return pl.pallas_call(
        _norm_body,
        in_specs=[pl.BlockSpec((_NC, _NR // _D, _D), lambda: (0, 0, 0))],
        out_specs=pl.BlockSpec((_NR // _D, _D), lambda: (0, 0)),
        out_shape=jax.ShapeDtypeStruct((_NR // _D, _D), jnp.float32),
    )(degp)


_sc_mesh = plsc.VectorSubcoreMesh(core_axis_name="c", subcore_axis_name="s")


@functools.partial(
    pl.kernel,
    mesh=_sc_mesh,
    out_type=jax.ShapeDtypeStruct((_NC * _NR,), jnp.float32),
    scratch_types=[
        pltpu.VMEM_SHARED((_NR,), jnp.float32),
        pltpu.VMEM((_CHUNK,), jnp.int32),
        pltpu.VMEM((_CHUNK,), jnp.int32),
        pltpu.VMEM((_ROWS, _B), jnp.int32),
        pltpu.VMEM((_CHUNK,), jnp.float32),
        pltpu.VMEM((_B,), jnp.float32),
        pltpu.SemaphoreType.DMA,
    ],
)
def _sc_deg(rel_hbm, dst_hbm, out_hbm,
            deg_sh, relv, dstv, p1pair, degv, ones, sem_s):
    c = lax.axis_index("c")
    s = lax.axis_index("s")

    def zero_vec(j, carry):
        degv[pl.ds(j * 16, 16)] = jnp.zeros((16,), jnp.float32)
        return carry

    lax.fori_loop(0, _CHUNK // 16, zero_vec, 0)

    def zero_deg(k, carry):
        pltpu.sync_copy(degv, deg_sh.at[pl.ds(s * _DEG_T + k * _CHUNK, _CHUNK)])
        return carry

    lax.fori_loop(0, _DEG_T // _CHUNK, zero_deg, 0)
    for q in range(_B // 16):
        ones[pl.ds(q * 16, 16)] = jnp.full((16,), 1.0, jnp.float32)
    plsc.subcore_barrier()

    def p1_chunk(k, carry):
        e0 = c * (_E // _NC) + s * _P3_EDGES + k * _CHUNK
        pltpu.sync_copy(rel_hbm.at[pl.ds(e0, _CHUNK)], relv)
        pltpu.sync_copy(dst_hbm.at[pl.ds(e0, _CHUNK)], dstv)

        def p1_row(r, carry2):
            for q in range(_B // 16):
                o = r * _B + q * 16
                pr = dstv[pl.ds(o, 16)] * _R + relv[pl.ds(o, 16)]
                p1pair[r, pl.ds(q * 16, 16)] = pr
            return carry2

        lax.fori_loop(0, _ROWS, p1_row, 0)

        def p1_scat(r, carry2):
            pltpu.async_copy(ones, deg_sh.at[p1pair.at[r]], sem_s, add=True)
            return carry2

        lax.fori_loop(0, _ROWS, p1_scat, 0)

        def p1_drain(r, carry2):
            pltpu.make_async_copy(ones, deg_sh.at[p1pair.at[r]], sem_s).wait()
            return carry2

        lax.fori_loop(0, _ROWS, p1_drain, 0)
        return carry

    lax.fori_loop(0, _P3_EDGES // _CHUNK, p1_chunk, 0)
    plsc.subcore_barrier()

    def wb_chunk(k, carry):
        off = s * _DEG_T + k * _CHUNK
        pltpu.sync_copy(deg_sh.at[pl.ds(off, _CHUNK)], degv)
        pltpu.sync_copy(degv, out_hbm.at[pl.ds(c * _NR + off, _CHUNK)])
        return carry

    lax.fori_loop(0, _DEG_T // _CHUNK, wb_chunk, 0)


@functools.partial(
    pl.kernel,
    mesh=_sc_mesh,
    out_type=jax.ShapeDtypeStruct((_NC, _N, _D), jnp.float32),
    scratch_types=[
        pltpu.VMEM_SHARED((_N, _D), jnp.float32),
        pltpu.VMEM((_CHUNK,), jnp.int32),
        pltpu.VMEM((_CHUNK,), jnp.int32),
        pltpu.VMEM((_CHUNK,), jnp.int32),
        pltpu.VMEM((_ROWS, _B), jnp.int32),
        pltpu.VMEM((_ROWS, _B), jnp.int32),
        pltpu.VMEM((_ROWS, _B), jnp.int32),
        pltpu.VMEM((_ROWS, _B), jnp.float32),
        pltpu.VMEM((2, _B, _D), jnp.float32),
        pltpu.VMEM((_B,), jnp.float32),
        pltpu.SemaphoreType.DMA,
        pltpu.SemaphoreType.DMA,
        pltpu.SemaphoreType.DMA,
    ],
)
def _sc_msg(src_hbm, rel_hbm, dst_hbm, table_hbm, norm_hbm, zagg_hbm,
            out_hbm,
            agg_sh, relv, dstv, srcv, pair2, didx2, gidx2,
            norm2, rows2, ones, sem_g, sem_n, sem_s):
    c = lax.axis_index("c")
    s = lax.axis_index("s")

    @pl.when(s < _NS - 1)
    def _():
        pltpu.sync_copy(zagg_hbm.at[pl.ds(s * _NODES_T, _NODES_T)],
                        agg_sh.at[pl.ds(s * _NODES_T, _NODES_T)])

    @pl.when(s == _NS - 1)
    def _():
        pltpu.sync_copy(zagg_hbm.at[pl.ds((_NS - 1) * _NODES_T, _LAST_T)],
                        agg_sh.at[pl.ds((_NS - 1) * _NODES_T, _LAST_T)])
    plsc.subcore_barrier()

    def p3_chunk(k, carry):
        e0 = c * (_E // _NC) + s * _P3_EDGES + k * _CHUNK
        pltpu.sync_copy(src_hbm.at[pl.ds(e0, _CHUNK)], srcv)
        pltpu.sync_copy(rel_hbm.at[pl.ds(e0, _CHUNK)], relv)
        pltpu.sync_copy(dst_hbm.at[pl.ds(e0, _CHUNK)], dstv)

        def p3_idx(r, carry2):
            for q in range(_B // 16):
                o = r * _B + q * 16
                rl = relv[pl.ds(o, 16)]
                sr = srcv[pl.ds(o, 16)]
                dd = dstv[pl.ds(o, 16)]
                gidx2[r, pl.ds(q * 16, 16)] = rl * _N + sr
                pair2[r, pl.ds(q * 16, 16)] = dd * _R + rl
                didx2[r, pl.ds(q * 16, 16)] = dd
            return carry2

        lax.fori_loop(0, _ROWS, p3_idx, 0)

        def nfire(r, carry2):
            pltpu.async_copy(norm_hbm.at[pair2.at[r]], norm2.at[r], sem_n)
            return carry2

        lax.fori_loop(0, _ROWS, nfire, 0)

        def ndrain(r, carry2):
            pltpu.make_async_copy(norm_hbm.at[pair2.at[r]], norm2.at[r],
                                  sem_n).wait()
            return carry2

        lax.fori_loop(0, _ROWS, ndrain, 0)

        pltpu.async_copy(table_hbm.at[gidx2.at[0]], rows2.at[0], sem_g)

        def p3_row(r, carry2):
            b = lax.rem(r, 2)

            @pl.when(r >= 1)
            def _():
                pltpu.make_async_copy(rows2.at[1 - b], agg_sh.at[didx2.at[0]],
                                      sem_s).wait()

            @pl.when(r + 1 < _ROWS)
            def _():
                pltpu.async_copy(table_hbm.at[gidx2.at[r + 1]],
                                 rows2.at[1 - b], sem_g)

            pltpu.make_async_copy(table_hbm.at[gidx2.at[r]], rows2.at[b],
                                  sem_g).wait()

            def scale(g, carry3):
                n16 = norm2[r, pl.ds(g * 16, 16)]
                for l in range(16):
                    nb = jnp.full((16,), n16[l], jnp.float32)
                    j = g * 16 + l
                    for q in range(_D // 16):
                        v = rows2[b, j, pl.ds(q * 16, 16)]
                        rows2[b, j, pl.ds(q * 16, 16)] = v * nb
                return carry3

            lax.fori_loop(0, _B // 16, scale, 0)
            pltpu.async_copy(rows2.at[b], agg_sh.at[didx2.at[r]], sem_s,
                             add=True)
            return carry2

        lax.fori_loop(0, _ROWS, p3_row, 0)
        pltpu.make_async_copy(rows2.at[0], agg_sh.at[didx2.at[0]], sem_s).wait()
        return carry

    lax.fori_loop(0, _P3_EDGES // _CHUNK, p3_chunk, 0)
    plsc.subcore_barrier()

    @pl.when(s < _NS - 1)
    def _():
        pltpu.sync_copy(agg_sh.at[pl.ds(s * _NODES_T, _NODES_T)],
                        out_hbm.at[c, pl.ds(s * _NODES_T, _NODES_T)])

    @pl.when(s == _NS - 1)
    def _():
        pltpu.sync_copy(agg_sh.at[pl.ds((_NS - 1) * _NODES_T, _LAST_T)],
                        out_hbm.at[c, pl.ds((_NS - 1) * _NODES_T, _LAST_T)])


def kernel(edges, h_input, W_rel, W_self):
    src = edges[:, 0].astype(jnp.int32)
    rel = (edges[:, 1] % _R).astype(jnp.int32)
    dst = edges[:, 2].astype(jnp.int32)
    transformed = _transform(h_input, W_rel).reshape(_R * _N, _D)
    degp = _sc_deg(rel, dst).reshape(_NC, _NR // _D, _D)
    norm = _norm(degp).reshape(_NR)
    zagg = jnp.zeros((_N, _D), jnp.float32)
    partials = _sc_msg(src, rel, dst, transformed, norm, zagg)
    return _final(partials, h_input, W_self)


# speedup vs baseline: 1.4338x; 1.4010x over previous
import functools

import jax
import jax.numpy as jnp
from jax import lax
from jax.experimental import pallas as pl
from jax.experimental.pallas import tpu as pltpu
from jax.experimental.pallas import tpu_sc as plsc

_N = 10000
_R = 16
_D = 128
_E = 320000
_NR = _N * _R
_NC = 2
_NS = 16
_CHUNK = 2000
_B = 80
_ROWS = _CHUNK // _B
_P1_EDGES = _E // _NS
_P3_EDGES = _E // (_NC * _NS)
_NODES_T = 632
_LAST_T = _N - (_NS - 1) * _NODES_T
_DEG_T = _NR // _NS


def _transform_body(h_ref, w_ref, o_ref):
    o_ref[0] = jnp.dot(h_ref[...], w_ref[0], preferred_element_type=jnp.float32)


def _final_body(p_ref, h_ref, ws_ref, o_ref):
    o_ref[...] = jax.nn.relu(
        p_ref[0] + p_ref[1]
        + jnp.dot(h_ref[...], ws_ref[...], preferred_element_type=jnp.float32)
    )


_BN = 1000


def _transform(h, W_rel):
    return pl.pallas_call(
        _transform_body,
        grid=(_R, _N // _BN),
        in_specs=[
            pl.BlockSpec((_BN, _D), lambda r, nb: (nb, 0)),
            pl.BlockSpec((1, _D, _D), lambda r, nb: (r, 0, 0)),
        ],
        out_specs=pl.BlockSpec((1, _BN, _D), lambda r, nb: (r, nb, 0)),
        out_shape=jax.ShapeDtypeStruct((_R, _N, _D), jnp.float32),
    )(h, W_rel)


def _final(partials, h, W_self):
    return pl.pallas_call(
        _final_body,
        grid=(_N // _BN,),
        in_specs=[
            pl.BlockSpec((_NC, _BN, _D), lambda nb: (0, nb, 0)),
            pl.BlockSpec((_BN, _D), lambda nb: (nb, 0)),
            pl.BlockSpec((_D, _D), lambda nb: (0, 0)),
        ],
        out_specs=pl.BlockSpec((_BN, _D), lambda nb: (nb, 0)),
        out_shape=jax.ShapeDtypeStruct((_N, _D), jnp.float32),
    )(partials, h, W_self)


def _norm_body(d_ref, o_ref):
    o_ref[...] = 1.0 / jnp.maximum(d_ref[0] + d_ref[1], 1.0)


def _norm(degp):
    return pl.pallas_call(
        _norm_body,
        in_specs=[pl.BlockSpec((_NC, _NR // _D, _D), lambda: (0, 0, 0))],
        out_specs=pl.BlockSpec((_NR // _D, _D), lambda: (0, 0)),
        out_shape=jax.ShapeDtypeStruct((_NR // _D, _D), jnp.float32),
    )(degp)


_sc_mesh = plsc.VectorSubcoreMesh(core_axis_name="c", subcore_axis_name="s")


@functools.partial(
    pl.kernel,
    mesh=_sc_mesh,
    out_type=jax.ShapeDtypeStruct((_NC * _NR,), jnp.float32),
    scratch_types=[
        pltpu.VMEM_SHARED((_NR,), jnp.float32),
        pltpu.VMEM((_CHUNK,), jnp.int32),
        pltpu.VMEM((_CHUNK,), jnp.int32),
        pltpu.VMEM((_ROWS, _B), jnp.int32),
        pltpu.VMEM((_CHUNK,), jnp.float32),
        pltpu.VMEM((_B,), jnp.float32),
        pltpu.SemaphoreType.DMA,
    ],
)
def _sc_deg(rel_hbm, dst_hbm, out_hbm,
            deg_sh, relv, dstv, p1pair, degv, ones, sem_s):
    c = lax.axis_index("c")
    s = lax.axis_index("s")

    def zero_vec(j, carry):
        degv[pl.ds(j * 16, 16)] = jnp.zeros((16,), jnp.float32)
        return carry

    lax.fori_loop(0, _CHUNK // 16, zero_vec, 0)

    def zero_deg(k, carry):
        pltpu.sync_copy(degv, deg_sh.at[pl.ds(s * _DEG_T + k * _CHUNK, _CHUNK)])
        return carry

    lax.fori_loop(0, _DEG_T // _CHUNK, zero_deg, 0)
    for q in range(_B // 16):
        ones[pl.ds(q * 16, 16)] = jnp.full((16,), 1.0, jnp.float32)
    plsc.subcore_barrier()

    def p1_chunk(k, carry):
        e0 = c * (_E // _NC) + s * _P3_EDGES + k * _CHUNK
        pltpu.sync_copy(rel_hbm.at[pl.ds(e0, _CHUNK)], relv)
        pltpu.sync_copy(dst_hbm.at[pl.ds(e0, _CHUNK)], dstv)

        def p1_row(r, carry2):
            for q in range(_B // 16):
                o = r * _B + q * 16
                pr = dstv[pl.ds(o, 16)] * _R + relv[pl.ds(o, 16)]
                p1pair[r, pl.ds(q * 16, 16)] = pr
            return carry2

        lax.fori_loop(0, _ROWS, p1_row, 0)

        def p1_scat(r, carry2):
            pltpu.async_copy(ones, deg_sh.at[p1pair.at[r]], sem_s, add=True)
            return carry2

        lax.fori_loop(0, _ROWS, p1_scat, 0)

        def p1_drain(r, carry2):
            pltpu.make_async_copy(ones, deg_sh.at[p1pair.at[r]], sem_s).wait()
            return carry2

        lax.fori_loop(0, _ROWS, p1_drain, 0)
        return carry

    lax.fori_loop(0, _P3_EDGES // _CHUNK, p1_chunk, 0)
    plsc.subcore_barrier()

    def wb_chunk(k, carry):
        off = s * _DEG_T + k * _CHUNK
        pltpu.sync_copy(deg_sh.at[pl.ds(off, _CHUNK)], degv)
        pltpu.sync_copy(degv, out_hbm.at[pl.ds(c * _NR + off, _CHUNK)])
        return carry

    lax.fori_loop(0, _DEG_T // _CHUNK, wb_chunk, 0)


@functools.partial(
    pl.kernel,
    mesh=_sc_mesh,
    out_type=jax.ShapeDtypeStruct((_NC, _N, _D), jnp.float32),
    scratch_types=[
        pltpu.VMEM_SHARED((_N, _D), jnp.float32),
        pltpu.VMEM((_CHUNK,), jnp.int32),
        pltpu.VMEM((_CHUNK,), jnp.int32),
        pltpu.VMEM((_CHUNK,), jnp.int32),
        pltpu.VMEM((_ROWS, _B), jnp.int32),
        pltpu.VMEM((_ROWS, _B), jnp.int32),
        pltpu.VMEM((_ROWS, _B), jnp.int32),
        pltpu.VMEM((_ROWS, _B), jnp.float32),
        pltpu.VMEM((2, _B, _D), jnp.float32),
        pltpu.VMEM((_B,), jnp.float32),
        pltpu.SemaphoreType.DMA,
        pltpu.SemaphoreType.DMA,
        pltpu.SemaphoreType.DMA,
    ],
)
def _sc_msg(src_hbm, rel_hbm, dst_hbm, table_hbm, norm_hbm, zagg_hbm,
            out_hbm,
            agg_sh, relv, dstv, srcv, pair2, didx2, gidx2,
            norm2, rows2, ones, sem_g, sem_n, sem_s):
    c = lax.axis_index("c")
    s = lax.axis_index("s")

    @pl.when(s < _NS - 1)
    def _():
        pltpu.sync_copy(zagg_hbm.at[pl.ds(s * _NODES_T, _NODES_T)],
                        agg_sh.at[pl.ds(s * _NODES_T, _NODES_T)])

    @pl.when(s == _NS - 1)
    def _():
        pltpu.sync_copy(zagg_hbm.at[pl.ds((_NS - 1) * _NODES_T, _LAST_T)],
                        agg_sh.at[pl.ds((_NS - 1) * _NODES_T, _LAST_T)])
    plsc.subcore_barrier()

    def p3_chunk(k, carry):
        e0 = c * (_E // _NC) + s * _P3_EDGES + k * _CHUNK
        pltpu.sync_copy(src_hbm.at[pl.ds(e0, _CHUNK)], srcv)
        pltpu.sync_copy(rel_hbm.at[pl.ds(e0, _CHUNK)], relv)
        pltpu.sync_copy(dst_hbm.at[pl.ds(e0, _CHUNK)], dstv)

        def p3_idx(r, carry2):
            for q in range(_B // 16):
                o = r * _B + q * 16
                rl = relv[pl.ds(o, 16)]
                sr = srcv[pl.ds(o, 16)]
                dd = dstv[pl.ds(o, 16)]
                gidx2[r, pl.ds(q * 16, 16)] = rl * _N + sr
                pair2[r, pl.ds(q * 16, 16)] = dd * _R + rl
                didx2[r, pl.ds(q * 16, 16)] = dd
            return carry2

        lax.fori_loop(0, _ROWS, p3_idx, 0)

        def nfire(r, carry2):
            pltpu.async_copy(norm_hbm.at[pair2.at[r]], norm2.at[r], sem_n)
            return carry2

        lax.fori_loop(0, _ROWS, nfire, 0)

        def ndrain(r, carry2):
            pltpu.make_async_copy(norm_hbm.at[pair2.at[r]], norm2.at[r],
                                  sem_n).wait()
            return carry2

        lax.fori_loop(0, _ROWS, ndrain, 0)

        def p3_row(r, carry2):
            pltpu.async_copy(table_hbm.at[gidx2.at[r]], rows2.at[0],
                             sem_g).wait()

            def scale(g, carry3):
                n16 = norm2[r, pl.ds(g * 16, 16)]
                for l in range(16):
                    nb = jnp.full((16,), n16[l], jnp.float32)
                    j = g * 16 + l
                    for q in range(_D // 16):
                        v = rows2[0, j, pl.ds(q * 16, 16)]
                        rows2[0, j, pl.ds(q * 16, 16)] = v * nb
                return carry3

            lax.fori_loop(0, _B // 16, scale, 0)
            pltpu.sync_copy(rows2.at[0], agg_sh.at[didx2.at[r]], add=True)
            return carry2

        lax.fori_loop(0, _ROWS, p3_row, 0)
        return carry

    lax.fori_loop(0, _P3_EDGES // _CHUNK, p3_chunk, 0)
    plsc.subcore_barrier()

    @pl.when(s < _NS - 1)
    def _():
        pltpu.sync_copy(agg_sh.at[pl.ds(s * _NODES_T, _NODES_T)],
                        out_hbm.at[c, pl.ds(s * _NODES_T, _NODES_T)])

    @pl.when(s == _NS - 1)
    def _():
        pltpu.sync_copy(agg_sh.at[pl.ds((_NS - 1) * _NODES_T, _LAST_T)],
                        out_hbm.at[c, pl.ds((_NS - 1) * _NODES_T, _LAST_T)])


def kernel(edges, h_input, W_rel, W_self):
    src = edges[:, 0].astype(jnp.int32)
    rel = (edges[:, 1] % _R).astype(jnp.int32)
    dst = edges[:, 2].astype(jnp.int32)
    transformed = _transform(h_input, W_rel).reshape(_R * _N, _D)
    degp = _sc_deg(rel, dst).reshape(_NC, _NR // _D, _D)
    norm = _norm(degp).reshape(_NR)
    zagg = jnp.zeros((_N, _D), jnp.float32)
    partials = _sc_msg(src, rel, dst, transformed, norm, zagg)
    return _final(partials, h_input, W_self)


# speedup vs baseline: 1.8036x; 1.2579x over previous
import functools

import jax
import jax.numpy as jnp
from jax import lax
from jax.experimental import pallas as pl
from jax.experimental.pallas import tpu as pltpu
from jax.experimental.pallas import tpu_sc as plsc

_N = 10000
_R = 16
_D = 128
_E = 320000
_NR = _N * _R
_NC = 2
_NS = 16
_CHUNK = 2000
_B = 80
_ROWS = _CHUNK // _B
_P1_EDGES = _E // _NS
_P3_EDGES = _E // (_NC * _NS)
_NODES_T = 632
_LAST_T = _N - (_NS - 1) * _NODES_T
_DEG_T = _NR // _NS


def _transform_body(h_ref, w_ref, o_ref):
    o_ref[0] = jnp.dot(h_ref[...], w_ref[0], preferred_element_type=jnp.float32)


def _final_body(p_ref, h_ref, ws_ref, o_ref):
    o_ref[...] = jax.nn.relu(
        p_ref[0] + p_ref[1]
        + jnp.dot(h_ref[...], ws_ref[...], preferred_element_type=jnp.float32)
    )


_BN = 1000


def _transform(h, W_rel):
    return pl.pallas_call(
        _transform_body,
        grid=(_R, _N // _BN),
        in_specs=[
            pl.BlockSpec((_BN, _D), lambda r, nb: (nb, 0)),
            pl.BlockSpec((1, _D, _D), lambda r, nb: (r, 0, 0)),
        ],
        out_specs=pl.BlockSpec((1, _BN, _D), lambda r, nb: (r, nb, 0)),
        out_shape=jax.ShapeDtypeStruct((_R, _N, _D), jnp.float32),
    )(h, W_rel)


def _final(partials, h, W_self):
    return pl.pallas_call(
        _final_body,
        grid=(_N // _BN,),
        in_specs=[
            pl.BlockSpec((_NC, _BN, _D), lambda nb: (0, nb, 0)),
            pl.BlockSpec((_BN, _D), lambda nb: (nb, 0)),
            pl.BlockSpec((_D, _D), lambda nb: (0, 0)),
        ],
        out_specs=pl.BlockSpec((_BN, _D), lambda nb: (nb, 0)),
        out_shape=jax.ShapeDtypeStruct((_N, _D), jnp.float32),
    )(partials, h, W_self)


def _norm_body(d_ref, o_ref):
    o_ref[...] = 1.0 / jnp.maximum(d_ref[0] + d_ref[1], 1.0)


def _norm(degp):
    return pl.pallas_call(
        _norm_body,
        in_specs=[pl.BlockSpec((_NC, _NR // _D, _D), lambda: (0, 0, 0))],
        out_specs=pl.BlockSpec((_NR // _D, _D), lambda: (0, 0)),
        out_shape=jax.ShapeDtypeStruct((_NR // _D, _D), jnp.float32),
    )(degp)


_sc_mesh = plsc.VectorSubcoreMesh(core_axis_name="c", subcore_axis_name="s")


@functools.partial(
    pl.kernel,
    mesh=_sc_mesh,
    out_type=jax.ShapeDtypeStruct((_NC * _NR,), jnp.float32),
    scratch_types=[
        pltpu.VMEM_SHARED((_NR,), jnp.float32),
        pltpu.VMEM((_CHUNK,), jnp.int32),
        pltpu.VMEM((_CHUNK,), jnp.int32),
        pltpu.VMEM((_ROWS, _B), jnp.int32),
        pltpu.VMEM((_CHUNK,), jnp.float32),
        pltpu.VMEM((_B,), jnp.float32),
        pltpu.SemaphoreType.DMA,
    ],
)
def _sc_deg(rel_hbm, dst_hbm, out_hbm,
            deg_sh, relv, dstv, p1pair, degv, ones, sem_s):
    c = lax.axis_index("c")
    s = lax.axis_index("s")

    def zero_vec(j, carry):
        degv[pl.ds(j * 16, 16)] = jnp.zeros((16,), jnp.float32)
        return carry

    lax.fori_loop(0, _CHUNK // 16, zero_vec, 0)

    def zero_deg(k, carry):
        pltpu.sync_copy(degv, deg_sh.at[pl.ds(s * _DEG_T + k * _CHUNK, _CHUNK)])
        return carry

    lax.fori_loop(0, _DEG_T // _CHUNK, zero_deg, 0)
    for q in range(_B // 16):
        ones[pl.ds(q * 16, 16)] = jnp.full((16,), 1.0, jnp.float32)
    plsc.subcore_barrier()

    def p1_chunk(k, carry):
        e0 = c * (_E // _NC) + s * _P3_EDGES + k * _CHUNK
        pltpu.sync_copy(rel_hbm.at[pl.ds(e0, _CHUNK)], relv)
        pltpu.sync_copy(dst_hbm.at[pl.ds(e0, _CHUNK)], dstv)

        def p1_row(r, carry2):
            for q in range(_B // 16):
                o = r * _B + q * 16
                pr = dstv[pl.ds(o, 16)] * _R + relv[pl.ds(o, 16)]
                p1pair[r, pl.ds(q * 16, 16)] = pr
            return carry2

        lax.fori_loop(0, _ROWS, p1_row, 0)

        def p1_scat(r, carry2):
            pltpu.async_copy(ones, deg_sh.at[p1pair.at[r]], sem_s, add=True)
            return carry2

        lax.fori_loop(0, _ROWS, p1_scat, 0)

        def p1_drain(r, carry2):
            pltpu.make_async_copy(ones, deg_sh.at[p1pair.at[r]], sem_s).wait()
            return carry2

        lax.fori_loop(0, _ROWS, p1_drain, 0)
        return carry

    lax.fori_loop(0, _P3_EDGES // _CHUNK, p1_chunk, 0)
    plsc.subcore_barrier()

    def wb_chunk(k, carry):
        off = s * _DEG_T + k * _CHUNK
        pltpu.sync_copy(deg_sh.at[pl.ds(off, _CHUNK)], degv)
        pltpu.sync_copy(degv, out_hbm.at[pl.ds(c * _NR + off, _CHUNK)])
        return carry

    lax.fori_loop(0, _DEG_T // _CHUNK, wb_chunk, 0)


@functools.partial(
    pl.kernel,
    mesh=_sc_mesh,
    out_type=jax.ShapeDtypeStruct((_NC, _N, _D), jnp.float32),
    scratch_types=[
        pltpu.VMEM_SHARED((_N, _D), jnp.float32),
        pltpu.VMEM((_CHUNK,), jnp.int32),
        pltpu.VMEM((_CHUNK,), jnp.int32),
        pltpu.VMEM((_CHUNK,), jnp.int32),
        pltpu.VMEM((_ROWS, _B), jnp.int32),
        pltpu.VMEM((_ROWS, _B), jnp.int32),
        pltpu.VMEM((_ROWS, _B), jnp.int32),
        pltpu.VMEM((_ROWS, _B), jnp.float32),
        pltpu.VMEM((2, _B, _D), jnp.float32),
        pltpu.VMEM((_B,), jnp.float32),
        pltpu.SemaphoreType.DMA,
        pltpu.SemaphoreType.DMA,
        pltpu.SemaphoreType.DMA,
    ],
)
def _sc_msg(src_hbm, rel_hbm, dst_hbm, table_hbm, norm_hbm, zagg_hbm,
            out_hbm,
            agg_sh, relv, dstv, srcv, pair2, didx2, gidx2,
            norm2, rows2, ones, sem_g, sem_n, sem_s):
    c = lax.axis_index("c")
    s = lax.axis_index("s")

    @pl.when(s < _NS - 1)
    def _():
        pltpu.sync_copy(zagg_hbm.at[pl.ds(s * _NODES_T, _NODES_T)],
                        agg_sh.at[pl.ds(s * _NODES_T, _NODES_T)])

    @pl.when(s == _NS - 1)
    def _():
        pltpu.sync_copy(zagg_hbm.at[pl.ds((_NS - 1) * _NODES_T, _LAST_T)],
                        agg_sh.at[pl.ds((_NS - 1) * _NODES_T, _LAST_T)])
    plsc.subcore_barrier()

    def p3_chunk(k, carry):
        e0 = c * (_E // _NC) + s * _P3_EDGES + k * _CHUNK
        pltpu.sync_copy(src_hbm.at[pl.ds(e0, _CHUNK)], srcv)
        pltpu.sync_copy(rel_hbm.at[pl.ds(e0, _CHUNK)], relv)
        pltpu.sync_copy(dst_hbm.at[pl.ds(e0, _CHUNK)], dstv)

        def p3_idx(r, carry2):
            for q in range(_B // 16):
                o = r * _B + q * 16
                rl = relv[pl.ds(o, 16)]
                sr = srcv[pl.ds(o, 16)]
                dd = dstv[pl.ds(o, 16)]
                gidx2[r, pl.ds(q * 16, 16)] = rl * _N + sr
                pair2[r, pl.ds(q * 16, 16)] = dd * _R + rl
                didx2[r, pl.ds(q * 16, 16)] = dd
            return carry2

        lax.fori_loop(0, _ROWS, p3_idx, 0)

        def nfire(r, carry2):
            pltpu.async_copy(norm_hbm.at[pair2.at[r]], norm2.at[r], sem_n)
            return carry2

        lax.fori_loop(0, _ROWS, nfire, 0)

        def ndrain(r, carry2):
            pltpu.make_async_copy(norm_hbm.at[pair2.at[r]], norm2.at[r],
                                  sem_n).wait()
            return carry2

        lax.fori_loop(0, _ROWS, ndrain, 0)

        def do_row(r, buf):
            def scale(g, carry3):
                n16 = norm2[r, pl.ds(g * 16, 16)]
                for l in range(16):
                    nb = jnp.full((16,), n16[l], jnp.float32)
                    j = g * 16 + l
                    for q in range(_D // 16):
                        v = rows2[buf, j, pl.ds(q * 16, 16)]
                        rows2[buf, j, pl.ds(q * 16, 16)] = v * nb
                return carry3

            lax.fori_loop(0, _B // 16, scale, 0)
            pltpu.sync_copy(rows2.at[buf], agg_sh.at[didx2.at[r]], add=True)

        pltpu.async_copy(table_hbm.at[gidx2.at[0]], rows2.at[0], sem_g)

        def p3_pair(t, carry2):
            r0 = 2 * t
            r1 = r0 + 1
            pltpu.make_async_copy(table_hbm.at[gidx2.at[r0]], rows2.at[0],
                                  sem_g).wait()
            pltpu.async_copy(table_hbm.at[gidx2.at[r1]], rows2.at[1], sem_g)
            do_row(r0, 0)
            pltpu.make_async_copy(table_hbm.at[gidx2.at[r1]], rows2.at[1],
                                  sem_g).wait()
            pltpu.async_copy(table_hbm.at[gidx2.at[r1 + 1]], rows2.at[0], sem_g)
            do_row(r1, 1)
            return carry2

        lax.fori_loop(0, (_ROWS - 1) // 2, p3_pair, 0)
        pltpu.make_async_copy(table_hbm.at[gidx2.at[_ROWS - 1]], rows2.at[0],
                              sem_g).wait()
        do_row(_ROWS - 1, 0)
        return carry

    lax.fori_loop(0, _P3_EDGES // _CHUNK, p3_chunk, 0)
    plsc.subcore_barrier()

    @pl.when(s < _NS - 1)
    def _():
        pltpu.sync_copy(agg_sh.at[pl.ds(s * _NODES_T, _NODES_T)],
                        out_hbm.at[c, pl.ds(s * _NODES_T, _NODES_T)])

    @pl.when(s == _NS - 1)
    def _():
        pltpu.sync_copy(agg_sh.at[pl.ds((_NS - 1) * _NODES_T, _LAST_T)],
                        out_hbm.at[c, pl.ds((_NS - 1) * _NODES_T, _LAST_T)])


def kernel(edges, h_input, W_rel, W_self):
    src = edges[:, 0].astype(jnp.int32)
    rel = (edges[:, 1] % _R).astype(jnp.int32)
    dst = edges[:, 2].astype(jnp.int32)
    transformed = _transform(h_input, W_rel).reshape(_R * _N, _D)
    degp = _sc_deg(rel, dst).reshape(_NC, _NR // _D, _D)
    norm = _norm(degp).reshape(_NR)
    zagg = jnp.zeros((_N, _D), jnp.float32)
    partials = _sc_msg(src, rel, dst, transformed, norm, zagg)
    return _final(partials, h_input, W_self)
